# baseline (speedup 1.0000x reference)
import functools

import jax
import jax.numpy as jnp
from jax import lax
from jax.experimental import pallas as pl
from jax.experimental.pallas import tpu as pltpu
from jax.experimental.pallas import tpu_sc as plsc

N = 10000
FT = 128
NNZ = 160000
NCLS = 40
NG = 64

NC, NS = 2, 16
NW = NC * NS
PPT = NNZ // NW
CHUNK = 128
NCH = (PPT + CHUNK - 1) // CHUNK
SROWS = 10240

f32 = jnp.float32
i32 = jnp.int32

_mesh = plsc.VectorSubcoreMesh(core_axis_name="c", subcore_axis_name="s",
                               num_cores=NC, num_subcores=NS)


def _make_seg(W, nbuf=2):
    def body(src, gix, six, z2d, out, gv, sv, rows, g0, g1, acc):
        gsem = [g0, g1]
        cid = lax.axis_index("c")
        sid = lax.axis_index("s")
        wid = cid * NS + sid
        pltpu.sync_copy(gix.at[wid], gv)
        pltpu.sync_copy(six.at[wid], sv)
        rpt = SROWS // NS
        pltpu.sync_copy(z2d.at[pl.ds(0, rpt)], acc.at[pl.ds(sid * rpt, rpt)])
        plsc.subcore_barrier()
        pltpu.async_copy(src.at[gv.at[0]], rows.at[0], gsem[0])
        pltpu.async_copy(src.at[gv.at[1]], rows.at[1], gsem[1])

        @pl.loop(0, NCH, step=nbuf)
        def _grp(c):
            for b in range(nbuf):
                ch = c + b
                pltpu.make_async_copy(src.at[gv.at[ch]], rows.at[b],
                                      gsem[b]).wait()
                pltpu.sync_copy(rows.at[b], acc.at[sv.at[ch]], add=True)

                @pl.when(ch + 2 < NCH)
                def _():
                    pltpu.async_copy(src.at[gv.at[ch + 2]], rows.at[b],
                                     gsem[b])

        plsc.subcore_barrier()
        pltpu.sync_copy(acc.at[pl.ds(sid * rpt, rpt)],
                        out.at[cid, pl.ds(sid * rpt, rpt)])

    return pl.kernel(
        body,
        out_type=jax.ShapeDtypeStruct((NC, SROWS, W), f32),
        mesh=_mesh,
        scratch_types=(
            [pltpu.VMEM((NCH, CHUNK), i32),
             pltpu.VMEM((NCH, CHUNK), i32),
             pltpu.VMEM((nbuf, CHUNK, W), f32)]
            + [pltpu.SemaphoreType.DMA] * nbuf
            + [pltpu.VMEM_SHARED((SROWS, W), f32)]
        ),
    )


_seg = _make_seg(FT)

NCH1 = 79


@functools.partial(
    pl.kernel,
    out_type=jax.ShapeDtypeStruct((2 * SROWS,), f32),
    mesh=_mesh,
    scratch_types=[
        pltpu.VMEM((128,), f32),
        pltpu.VMEM((NCH1, CHUNK), i32),
        pltpu.VMEM((NCH, CHUNK), i32),
        pltpu.VMEM((NCH, CHUNK), i32),
        pltpu.VMEM((4, CHUNK), f32),
        pltpu.SemaphoreType.DMA,
        pltpu.SemaphoreType.DMA,
        pltpu.SemaphoreType.DMA,
        pltpu.SemaphoreType.DMA,
        pltpu.SemaphoreType.DMA,
        pltpu.SemaphoreType.DMA,
        pltpu.SemaphoreType.DMA,
        pltpu.SemaphoreType.DMA,
        pltpu.SemaphoreType.DMA,
        pltpu.VMEM_SHARED((SROWS,), f32),
        pltpu.VMEM_SHARED((SROWS,), f32),
    ],
)
def _deg(ones_in, s1e, gixe, sixn, z1, out, ov, s1v, gv, sv, gbuf,
         d0, g0, g1, g2, g3, s0, s1, s2, s3, aed, ad):
    cid = lax.axis_index("c")
    sid = lax.axis_index("s")
    wid = cid * NS + sid
    rpt = SROWS // NS
    pltpu.sync_copy(ones_in, ov)
    pltpu.sync_copy(s1e.at[sid], s1v)
    pltpu.sync_copy(gixe.at[wid], gv)
    pltpu.sync_copy(sixn.at[wid], sv)
    pltpu.sync_copy(z1, aed.at[pl.ds(sid * rpt, rpt)])
    pltpu.sync_copy(z1, ad.at[pl.ds(sid * rpt, rpt)])
    plsc.subcore_barrier()

    @pl.loop(0, NCH1)
    def _s1(ch):
        pltpu.async_copy(ov, aed.at[s1v.at[ch]], d0, add=True)

    @pl.loop(0, NCH1)
    def _s1d(ch):
        pltpu.make_async_copy(ov, aed.at[s1v.at[0]], d0).wait()

    plsc.subcore_barrier()
    gsem = [g0, g1, g2, g3]
    ssem = [s0, s1, s2, s3]
    pltpu.async_copy(aed.at[gv.at[0]], gbuf.at[0], gsem[0])
    pltpu.async_copy(aed.at[gv.at[1]], gbuf.at[1], gsem[1])

    @pl.loop(0, NCH, step=4)
    def _s2(c):
        for b in range(4):
            ch = c + b
            b2 = (b + 2) % 4
            pltpu.make_async_copy(aed.at[gv.at[ch]], gbuf.at[b],
                                  gsem[b]).wait()
            pltpu.async_copy(gbuf.at[b], ad.at[sv.at[ch]], ssem[b], add=True)

            @pl.when(ch + 2 < NCH)
            def _():
                @pl.when(ch >= 2)
                def _():
                    pltpu.make_async_copy(gbuf.at[b2], ad.at[sv.at[0]],
                                          ssem[b2]).wait()

                pltpu.async_copy(aed.at[gv.at[ch + 2]], gbuf.at[b2],
                                 gsem[b2])

    for b in range(4):
        pltpu.make_async_copy(gbuf.at[b], ad.at[sv.at[0]], ssem[b]).wait()
    plsc.subcore_barrier()
    pltpu.sync_copy(ad.at[pl.ds(sid * rpt, rpt)],
                    out.at[pl.ds(cid * SROWS + sid * rpt, rpt)])


BLK = 1000
GRID = N // BLK


def _row_spec(shape=(BLK, FT)):
    return pl.BlockSpec(shape, lambda i: (i, 0))


def _full_spec(shape):
    return pl.BlockSpec(shape, lambda i: (0, 0))


def _eadd(a, b):
    def body(ar, br, orf):
        orf[...] = ar[...] + br[...]

    spec = pl.BlockSpec((1024, FT), lambda i: (i, 0))
    return pl.pallas_call(
        body, grid=(SROWS // 1024,),
        in_specs=[spec, spec],
        out_specs=spec,
        out_shape=jax.ShapeDtypeStruct((SROWS, FT), f32),
    )(a, b)


def _mlp_a(y0, y1, W1, b1):
    def body(y0r, y1r, wr, br, sr, str_):
        i = pl.program_id(0)
        x = y0r[...] + y1r[...]
        s = jnp.dot(x, wr[...], preferred_element_type=f32) + br[...]
        sr[...] = s

        @pl.when(i == 0)
        def _():
            str_[...] = jnp.zeros_like(str_)

        str_[0:1, :] += jnp.sum(s, axis=0, keepdims=True)
        str_[1:2, :] += jnp.sum(s * s, axis=0, keepdims=True)

    return pl.pallas_call(
        body, grid=(GRID,),
        in_specs=[_row_spec(), _row_spec(), _full_spec((FT, FT)),
                  _full_spec((1, FT))],
        out_specs=[_row_spec(), _full_spec((8, FT))],
        out_shape=[jax.ShapeDtypeStruct((N, FT), f32),
                   jax.ShapeDtypeStruct((8, FT), f32)],
    )(y0, y1, W1, b1)


def _bn_coeffs(str_, g, b):
    mu = str_[0:1, :] * (1.0 / N)
    var = str_[1:2, :] * (1.0 / N) - mu * mu
    scale = g * lax.rsqrt(var + 1e-5)
    shift = b - mu * scale
    return scale, shift


def _mlp_b(S, st, g, b, W2, b2):
    def body(sr, stri, gr, br, wr, b2r, tr, stro):
        i = pl.program_id(0)
        scale, shift = _bn_coeffs(stri, gr[...], br[...])
        r = jnp.maximum(sr[...] * scale + shift, 0.0)
        t = jnp.dot(r, wr[...], preferred_element_type=f32) + b2r[...]
        tr[...] = t

        @pl.when(i == 0)
        def _():
            stro[...] = jnp.zeros_like(stro)

        stro[0:1, :] += jnp.sum(t, axis=0, keepdims=True)
        stro[1:2, :] += jnp.sum(t * t, axis=0, keepdims=True)

    return pl.pallas_call(
        body, grid=(GRID,),
        in_specs=[_row_spec(), _full_spec((8, FT)), _full_spec((1, FT)),
                  _full_spec((1, FT)), _full_spec((FT, FT)),
                  _full_spec((1, FT))],
        out_specs=[_row_spec(), _full_spec((8, FT))],
        out_shape=[jax.ShapeDtypeStruct((N, FT), f32),
                   jax.ShapeDtypeStruct((8, FT), f32)],
    )(S, st, g, b, W2, b2)


def _mlp_c(T, st, g, b):
    def body(tr, stri, gr, br, orf):
        scale, shift = _bn_coeffs(stri, gr[...], br[...])
        orf[...] = jnp.maximum(tr[...] * scale + shift, 0.0)

    return pl.pallas_call(
        body, grid=(GRID,),
        in_specs=[_row_spec(), _full_spec((8, FT)), _full_spec((1, FT)),
                  _full_spec((1, FT))],
        out_specs=_row_spec(),
        out_shape=jax.ShapeDtypeStruct((N, FT), f32),
    )(T, st, g, b)


def _head(p00, p01, p10, p11, d0r, d1r, bidr, Wh0, Wh1, bh):
    def body(a0, a1, c0, c1, dr0, dr1, idsr, w0, w1, bhr, outr, accr, cntr):
        i = pl.program_id(0)
        dv = dr0[0, 0, :BLK] + dr1[0, 0, :BLK]
        dsafe = jnp.where(dv == 0.0, 1.0, dv)
        dneg = jnp.where(dv == 0.0, 1.0, 1.0 / dsafe)
        h0 = (a0[...] + a1[...]) * dneg[:, None]
        h1 = (c0[...] + c1[...]) * dneg[:, None]
        o = (jnp.dot(h0, w0[...], preferred_element_type=f32)
             + jnp.dot(h1, w1[...], preferred_element_type=f32) + bhr[...])
        ids = idsr[0, 0, :BLK]
        oh = (lax.broadcasted_iota(i32, (NG, BLK), 0) == ids[None, :]
              ).astype(f32)

        @pl.when(i == 0)
        def _():
            accr[...] = jnp.zeros_like(accr)
            cntr[...] = jnp.zeros_like(cntr)

        accr[...] += jnp.dot(oh, o, preferred_element_type=f32)
        cntr[:, 0:1] += jnp.sum(oh, axis=1, keepdims=True)
        outr[...] = accr[...] / jnp.maximum(cntr[:, 0:1], 1.0)

    return pl.pallas_call(
        body, grid=(GRID,),
        in_specs=[_row_spec(), _row_spec(), _row_spec(), _row_spec(),
                  pl.BlockSpec((1, 1, 1024), lambda i: (i, 0, 0)),
                  pl.BlockSpec((1, 1, 1024), lambda i: (i, 0, 0)),
                  pl.BlockSpec((1, 1, 1024), lambda i: (i, 0, 0)),
                  _full_spec((FT, NCLS)), _full_spec((FT, NCLS)),
                  _full_spec((1, NCLS))],
        out_specs=_full_spec((NG, NCLS)),
        out_shape=jax.ShapeDtypeStruct((NG, NCLS), f32),
        scratch_shapes=[pltpu.VMEM((NG, NCLS), f32),
                        pltpu.VMEM((NG, 8), f32)],
    )(p00, p01, p10, p11, d0r, d1r, bidr, Wh0, Wh1, bh)


def _pad_pairs(idx, pad):
    r = idx.reshape(NW, PPT)
    return jnp.concatenate([r, pad], axis=1).reshape(NW, NCH, CHUNK)


def _pad_rows(v):
    return jnp.pad(v.reshape(GRID, BLK),
                   ((0, 0), (0, 1024 - BLK))).reshape(GRID, 1, 1024)


def kernel(X, params, H_node_idx, H_edge_idx, all_batch):
    p = params
    lane = jnp.arange(120, dtype=i32)[None, :]
    wrow = jnp.arange(NW, dtype=i32)[:, None]
    gpad = (lane * 97 + wrow * 53) % N
    spad = N + (lane + wrow) % 16
    gp_n = _pad_pairs(H_node_idx, gpad)
    sp_n = _pad_pairs(H_node_idx, spad)
    gp_e = _pad_pairs(H_edge_idx, gpad)
    sp_e = _pad_pairs(H_edge_idx, spad)
    z2d = jnp.zeros((SROWS // NS, FT), f32)
    z1 = jnp.zeros((SROWS // NS,), f32)
    dpad = SROWS - 224 + (lane[:, :112] + wrow[:16]) % 64
    s1e = jnp.concatenate([H_edge_idx.reshape(NS, NNZ // NS), dpad],
                          axis=1).reshape(NS, NCH1, CHUNK)

    bidr = _pad_rows(all_batch)
    b1_0 = p['b1_0'].reshape(1, FT); b2_0 = p['b2_0'].reshape(1, FT)
    b1_1 = p['b1_1'].reshape(1, FT); b2_1 = p['b2_1'].reshape(1, FT)
    bng_0 = p['bng_0'].reshape(1, FT); bnb_0 = p['bnb_0'].reshape(1, FT)
    bng_1 = p['bng_1'].reshape(1, FT); bnb_1 = p['bnb_1'].reshape(1, FT)
    g_e0 = p['g_e0'].reshape(1, FT); b_e0 = p['b_e0'].reshape(1, FT)
    g_e1 = p['g_e1'].reshape(1, FT); b_e1 = p['b_e1'].reshape(1, FT)
    Wh0 = p['Wh'][:FT]; Wh1 = p['Wh'][FT:]
    bh = p['bh'].reshape(1, NCLS)

    def prop(src):
        ep = _seg(src, gp_n, sp_e, z2d)
        ef = _eadd(ep[0], ep[1])
        return _seg(ef, gp_e, sp_n, z2d)

    dflat = _deg(jnp.ones((CHUNK,), f32), s1e, gp_e, sp_n, z1)
    d0r = _pad_rows(dflat[:N])
    d1r = _pad_rows(dflat[SROWS:SROWS + N])

    y0 = prop(X)
    S0, stA0 = _mlp_a(y0[0], y0[1], p['W1_0'], b1_0)
    T0, stB0 = _mlp_b(S0, stA0, bng_0, bnb_0, p['W2_0'], b2_0)
    h0 = _mlp_c(T0, stB0, g_e0, b_e0)
    y1 = prop(h0)
    S1, stA1 = _mlp_a(y1[0], y1[1], p['W1_1'], b1_1)
    T1, stB1 = _mlp_b(S1, stA1, bng_1, bnb_1, p['W2_1'], b2_1)
    h1 = _mlp_c(T1, stB1, g_e1, b_e1)
    r1 = prop(h1)
    return _head(y1[0], y1[1], r1[0], r1[1], d0r, d1r, bidr, Wh0, Wh1, bh)


# speedup vs baseline: 1.0311x; 1.0311x over previous
import functools

import jax
import jax.numpy as jnp
from jax import lax
from jax.experimental import pallas as pl
from jax.experimental.pallas import tpu as pltpu
from jax.experimental.pallas import tpu_sc as plsc

N = 10000
FT = 128
NNZ = 160000
NCLS = 40
NG = 64

NC, NS = 2, 16
NW = NC * NS
PPT = NNZ // NW
CHUNK = 128
NCH = (PPT + CHUNK - 1) // CHUNK
SROWS = 10240

f32 = jnp.float32
i32 = jnp.int32

_mesh = plsc.VectorSubcoreMesh(core_axis_name="c", subcore_axis_name="s",
                               num_cores=NC, num_subcores=NS)


def _make_seg(W, nbuf=2):
    def body(src, gix, six, z2d, out, gv, sv, rows, g0, g1, acc):
        gsem = [g0, g1]
        cid = lax.axis_index("c")
        sid = lax.axis_index("s")
        wid = cid * NS + sid
        pltpu.sync_copy(gix.at[wid], gv)
        pltpu.sync_copy(six.at[wid], sv)
        rpt = SROWS // NS
        pltpu.sync_copy(z2d.at[pl.ds(0, rpt)], acc.at[pl.ds(sid * rpt, rpt)])
        plsc.subcore_barrier()
        pltpu.async_copy(src.at[gv.at[0]], rows.at[0], gsem[0])
        pltpu.async_copy(src.at[gv.at[1]], rows.at[1], gsem[1])

        @pl.loop(0, NCH, step=nbuf)
        def _grp(c):
            for b in range(nbuf):
                ch = c + b
                pltpu.make_async_copy(src.at[gv.at[ch]], rows.at[b],
                                      gsem[b]).wait()
                pltpu.sync_copy(rows.at[b], acc.at[sv.at[ch]], add=True)

                @pl.when(ch + 2 < NCH)
                def _():
                    pltpu.async_copy(src.at[gv.at[ch + 2]], rows.at[b],
                                     gsem[b])

        plsc.subcore_barrier()
        pltpu.sync_copy(acc.at[pl.ds(sid * rpt, rpt)],
                        out.at[cid, pl.ds(sid * rpt, rpt)])

    return pl.kernel(
        body,
        out_type=jax.ShapeDtypeStruct((NC, SROWS, W), f32),
        mesh=_mesh,
        scratch_types=(
            [pltpu.VMEM((NCH, CHUNK), i32),
             pltpu.VMEM((NCH, CHUNK), i32),
             pltpu.VMEM((nbuf, CHUNK, W), f32)]
            + [pltpu.SemaphoreType.DMA] * nbuf
            + [pltpu.VMEM_SHARED((SROWS, W), f32)]
        ),
    )


_seg = _make_seg(FT)

NCH1 = 79


@functools.partial(
    pl.kernel,
    out_type=jax.ShapeDtypeStruct((2 * SROWS,), f32),
    mesh=_mesh,
    scratch_types=[
        pltpu.VMEM((128,), f32),
        pltpu.VMEM((NCH1, CHUNK), i32),
        pltpu.VMEM((NCH, CHUNK), i32),
        pltpu.VMEM((NCH, CHUNK), i32),
        pltpu.VMEM((4, CHUNK), f32),
        pltpu.SemaphoreType.DMA,
        pltpu.SemaphoreType.DMA,
        pltpu.SemaphoreType.DMA,
        pltpu.SemaphoreType.DMA,
        pltpu.SemaphoreType.DMA,
        pltpu.SemaphoreType.DMA,
        pltpu.SemaphoreType.DMA,
        pltpu.SemaphoreType.DMA,
        pltpu.SemaphoreType.DMA,
        pltpu.VMEM_SHARED((SROWS,), f32),
        pltpu.VMEM_SHARED((SROWS,), f32),
    ],
)
def _deg(ones_in, s1e, gixe, sixn, z1, out, ov, s1v, gv, sv, gbuf,
         d0, g0, g1, g2, g3, s0, s1, s2, s3, aed, ad):
    cid = lax.axis_index("c")
    sid = lax.axis_index("s")
    wid = cid * NS + sid
    rpt = SROWS // NS
    pltpu.sync_copy(ones_in, ov)
    pltpu.sync_copy(s1e.at[sid], s1v)
    pltpu.sync_copy(gixe.at[wid], gv)
    pltpu.sync_copy(sixn.at[wid], sv)
    pltpu.sync_copy(z1, aed.at[pl.ds(sid * rpt, rpt)])
    pltpu.sync_copy(z1, ad.at[pl.ds(sid * rpt, rpt)])
    plsc.subcore_barrier()

    @pl.loop(0, NCH1)
    def _s1(ch):
        pltpu.async_copy(ov, aed.at[s1v.at[ch]], d0, add=True)

    @pl.loop(0, NCH1)
    def _s1d(ch):
        pltpu.make_async_copy(ov, aed.at[s1v.at[0]], d0).wait()

    plsc.subcore_barrier()
    gsem = [g0, g1, g2, g3]
    ssem = [s0, s1, s2, s3]
    pltpu.async_copy(aed.at[gv.at[0]], gbuf.at[0], gsem[0])
    pltpu.async_copy(aed.at[gv.at[1]], gbuf.at[1], gsem[1])

    @pl.loop(0, NCH, step=4)
    def _s2(c):
        for b in range(4):
            ch = c + b
            b2 = (b + 2) % 4
            pltpu.make_async_copy(aed.at[gv.at[ch]], gbuf.at[b],
                                  gsem[b]).wait()
            pltpu.async_copy(gbuf.at[b], ad.at[sv.at[ch]], ssem[b], add=True)

            @pl.when(ch + 2 < NCH)
            def _():
                @pl.when(ch >= 2)
                def _():
                    pltpu.make_async_copy(gbuf.at[b2], ad.at[sv.at[0]],
                                          ssem[b2]).wait()

                pltpu.async_copy(aed.at[gv.at[ch + 2]], gbuf.at[b2],
                                 gsem[b2])

    for b in range(4):
        pltpu.make_async_copy(gbuf.at[b], ad.at[sv.at[0]], ssem[b]).wait()
    plsc.subcore_barrier()
    pltpu.sync_copy(ad.at[pl.ds(sid * rpt, rpt)],
                    out.at[pl.ds(cid * SROWS + sid * rpt, rpt)])


BLK = 2000
GRID = N // BLK


def _row_spec(shape=(BLK, FT)):
    return pl.BlockSpec(shape, lambda i: (i, 0))


def _full_spec(shape):
    return pl.BlockSpec(shape, lambda i: (0, 0))


def _eadd(a, b):
    def body(ar, br, orf):
        orf[...] = ar[...] + br[...]

    spec = pl.BlockSpec((1024, FT), lambda i: (i, 0))
    return pl.pallas_call(
        body, grid=(SROWS // 1024,),
        in_specs=[spec, spec],
        out_specs=spec,
        out_shape=jax.ShapeDtypeStruct((SROWS, FT), f32),
    )(a, b)


def _mlp_a(y0, y1, W1, b1):
    def body(y0r, y1r, wr, br, sr, str_):
        i = pl.program_id(0)
        x = y0r[...] + y1r[...]
        s = jnp.dot(x, wr[...], preferred_element_type=f32) + br[...]
        sr[...] = s

        @pl.when(i == 0)
        def _():
            str_[...] = jnp.zeros_like(str_)

        str_[0:1, :] += jnp.sum(s, axis=0, keepdims=True)
        str_[1:2, :] += jnp.sum(s * s, axis=0, keepdims=True)

    return pl.pallas_call(
        body, grid=(GRID,),
        in_specs=[_row_spec(), _row_spec(), _full_spec((FT, FT)),
                  _full_spec((1, FT))],
        out_specs=[_row_spec(), _full_spec((8, FT))],
        out_shape=[jax.ShapeDtypeStruct((N, FT), f32),
                   jax.ShapeDtypeStruct((8, FT), f32)],
    )(y0, y1, W1, b1)


def _bn_coeffs(str_, g, b):
    mu = str_[0:1, :] * (1.0 / N)
    var = str_[1:2, :] * (1.0 / N) - mu * mu
    scale = g * lax.rsqrt(var + 1e-5)
    shift = b - mu * scale
    return scale, shift


def _mlp_b(S, st, g, b, W2, b2):
    def body(sr, stri, gr, br, wr, b2r, tr, stro):
        i = pl.program_id(0)
        scale, shift = _bn_coeffs(stri, gr[...], br[...])
        r = jnp.maximum(sr[...] * scale + shift, 0.0)
        t = jnp.dot(r, wr[...], preferred_element_type=f32) + b2r[...]
        tr[...] = t

        @pl.when(i == 0)
        def _():
            stro[...] = jnp.zeros_like(stro)

        stro[0:1, :] += jnp.sum(t, axis=0, keepdims=True)
        stro[1:2, :] += jnp.sum(t * t, axis=0, keepdims=True)

    return pl.pallas_call(
        body, grid=(GRID,),
        in_specs=[_row_spec(), _full_spec((8, FT)), _full_spec((1, FT)),
                  _full_spec((1, FT)), _full_spec((FT, FT)),
                  _full_spec((1, FT))],
        out_specs=[_row_spec(), _full_spec((8, FT))],
        out_shape=[jax.ShapeDtypeStruct((N, FT), f32),
                   jax.ShapeDtypeStruct((8, FT), f32)],
    )(S, st, g, b, W2, b2)


def _mlp_c(T, st, g, b):
    def body(tr, stri, gr, br, orf):
        scale, shift = _bn_coeffs(stri, gr[...], br[...])
        orf[...] = jnp.maximum(tr[...] * scale + shift, 0.0)

    return pl.pallas_call(
        body, grid=(GRID,),
        in_specs=[_row_spec(), _full_spec((8, FT)), _full_spec((1, FT)),
                  _full_spec((1, FT))],
        out_specs=_row_spec(),
        out_shape=jax.ShapeDtypeStruct((N, FT), f32),
    )(T, st, g, b)


def _head(p00, p01, p10, p11, d0r, d1r, bidr, Wh0, Wh1, bh):
    def body(a0, a1, c0, c1, dr0, dr1, idsr, w0, w1, bhr, outr, accr, cntr):
        i = pl.program_id(0)
        dv = dr0[0, 0, :BLK] + dr1[0, 0, :BLK]
        dsafe = jnp.where(dv == 0.0, 1.0, dv)
        dneg = jnp.where(dv == 0.0, 1.0, 1.0 / dsafe)
        h0 = (a0[...] + a1[...]) * dneg[:, None]
        h1 = (c0[...] + c1[...]) * dneg[:, None]
        o = (jnp.dot(h0, w0[...], preferred_element_type=f32)
             + jnp.dot(h1, w1[...], preferred_element_type=f32) + bhr[...])
        ids = idsr[0, 0, :BLK]
        oh = (lax.broadcasted_iota(i32, (NG, BLK), 0) == ids[None, :]
              ).astype(f32)

        @pl.when(i == 0)
        def _():
            accr[...] = jnp.zeros_like(accr)
            cntr[...] = jnp.zeros_like(cntr)

        accr[...] += jnp.dot(oh, o, preferred_element_type=f32)
        cntr[:, 0:1] += jnp.sum(oh, axis=1, keepdims=True)
        outr[...] = accr[...] / jnp.maximum(cntr[:, 0:1], 1.0)

    return pl.pallas_call(
        body, grid=(GRID,),
        in_specs=[_row_spec(), _row_spec(), _row_spec(), _row_spec(),
                  pl.BlockSpec((1, 1, PADW), lambda i: (i, 0, 0)),
                  pl.BlockSpec((1, 1, PADW), lambda i: (i, 0, 0)),
                  pl.BlockSpec((1, 1, PADW), lambda i: (i, 0, 0)),
                  _full_spec((FT, NCLS)), _full_spec((FT, NCLS)),
                  _full_spec((1, NCLS))],
        out_specs=_full_spec((NG, NCLS)),
        out_shape=jax.ShapeDtypeStruct((NG, NCLS), f32),
        scratch_shapes=[pltpu.VMEM((NG, NCLS), f32),
                        pltpu.VMEM((NG, 8), f32)],
    )(p00, p01, p10, p11, d0r, d1r, bidr, Wh0, Wh1, bh)


def _pad_pairs(idx, pad):
    r = idx.reshape(NW, PPT)
    return jnp.concatenate([r, pad], axis=1).reshape(NW, NCH, CHUNK)


PADW = 2048


def _pad_rows(v):
    return jnp.pad(v.reshape(GRID, BLK),
                   ((0, 0), (0, PADW - BLK))).reshape(GRID, 1, PADW)


def kernel(X, params, H_node_idx, H_edge_idx, all_batch):
    p = params
    lane = jnp.arange(120, dtype=i32)[None, :]
    wrow = jnp.arange(NW, dtype=i32)[:, None]
    gpad = (lane * 97 + wrow * 53) % N
    spad = N + (lane * 7 + wrow * 11) % 240
    gp_n = _pad_pairs(H_node_idx, gpad)
    sp_n = _pad_pairs(H_node_idx, spad)
    gp_e = _pad_pairs(H_edge_idx, gpad)
    sp_e = _pad_pairs(H_edge_idx, spad)
    z2d = jnp.zeros((SROWS // NS, FT), f32)
    z1 = jnp.zeros((SROWS // NS,), f32)
    dpad = SROWS - 224 + (lane[:, :112] + wrow[:16]) % 64
    s1e = jnp.concatenate([H_edge_idx.reshape(NS, NNZ // NS), dpad],
                          axis=1).reshape(NS, NCH1, CHUNK)

    bidr = _pad_rows(all_batch)
    b1_0 = p['b1_0'].reshape(1, FT); b2_0 = p['b2_0'].reshape(1, FT)
    b1_1 = p['b1_1'].reshape(1, FT); b2_1 = p['b2_1'].reshape(1, FT)
    bng_0 = p['bng_0'].reshape(1, FT); bnb_0 = p['bnb_0'].reshape(1, FT)
    bng_1 = p['bng_1'].reshape(1, FT); bnb_1 = p['bnb_1'].reshape(1, FT)
    g_e0 = p['g_e0'].reshape(1, FT); b_e0 = p['b_e0'].reshape(1, FT)
    g_e1 = p['g_e1'].reshape(1, FT); b_e1 = p['b_e1'].reshape(1, FT)
    Wh0 = p['Wh'][:FT]; Wh1 = p['Wh'][FT:]
    bh = p['bh'].reshape(1, NCLS)

    def prop(src):
        ep = _seg(src, gp_n, sp_e, z2d)
        ef = _eadd(ep[0], ep[1])
        return _seg(ef, gp_e, sp_n, z2d)

    dflat = _deg(jnp.ones((CHUNK,), f32), s1e, gp_e, sp_n, z1)
    d0r = _pad_rows(dflat[:N])
    d1r = _pad_rows(dflat[SROWS:SROWS + N])

    y0 = prop(X)
    S0, stA0 = _mlp_a(y0[0], y0[1], p['W1_0'], b1_0)
    T0, stB0 = _mlp_b(S0, stA0, bng_0, bnb_0, p['W2_0'], b2_0)
    h0 = _mlp_c(T0, stB0, g_e0, b_e0)
    y1 = prop(h0)
    S1, stA1 = _mlp_a(y1[0], y1[1], p['W1_1'], b1_1)
    T1, stB1 = _mlp_b(S1, stA1, bng_1, bnb_1, p['W2_1'], b2_1)
    h1 = _mlp_c(T1, stB1, g_e1, b_e1)
    r1 = prop(h1)
    return _head(y1[0], y1[1], r1[0], r1[1], d0r, d1r, bidr, Wh0, Wh1, bh)


# speedup vs baseline: 1.0438x; 1.0123x over previous
import functools

import jax
import jax.numpy as jnp
from jax import lax
from jax.experimental import pallas as pl
from jax.experimental.pallas import tpu as pltpu
from jax.experimental.pallas import tpu_sc as plsc

N = 10000
FT = 128
NNZ = 160000
NCLS = 40
NG = 64

NC, NS = 2, 16
NW = NC * NS
PPT = NNZ // NW
CHUNK = 128
NCH = (PPT + CHUNK - 1) // CHUNK
SROWS = 10240

f32 = jnp.float32
i32 = jnp.int32

_mesh = plsc.VectorSubcoreMesh(core_axis_name="c", subcore_axis_name="s",
                               num_cores=NC, num_subcores=NS)


def _make_seg(W, nbuf=2):
    def body(src, gix, six, z2d, out, gv, sv, rows, g0, g1, acc):
        gsem = [g0, g1]
        cid = lax.axis_index("c")
        sid = lax.axis_index("s")
        wid = cid * NS + sid
        pltpu.sync_copy(gix.at[wid], gv)
        pltpu.sync_copy(six.at[wid], sv)
        rpt = SROWS // NS
        pltpu.sync_copy(z2d.at[pl.ds(0, rpt)], acc.at[pl.ds(sid * rpt, rpt)])
        plsc.subcore_barrier()
        pltpu.async_copy(src.at[gv.at[0]], rows.at[0], gsem[0])
        pltpu.async_copy(src.at[gv.at[1]], rows.at[1], gsem[1])

        @pl.loop(0, NCH, step=nbuf)
        def _grp(c):
            for b in range(nbuf):
                ch = c + b
                pltpu.make_async_copy(src.at[gv.at[ch]], rows.at[b],
                                      gsem[b]).wait()
                pltpu.sync_copy(rows.at[b], acc.at[sv.at[ch]], add=True)

                @pl.when(ch + 2 < NCH)
                def _():
                    pltpu.async_copy(src.at[gv.at[ch + 2]], rows.at[b],
                                     gsem[b])

        plsc.subcore_barrier()
        pltpu.sync_copy(acc.at[pl.ds(sid * rpt, rpt)],
                        out.at[cid, pl.ds(sid * rpt, rpt)])

    return pl.kernel(
        body,
        out_type=jax.ShapeDtypeStruct((NC, SROWS, W), f32),
        mesh=_mesh,
        scratch_types=(
            [pltpu.VMEM((NCH, CHUNK), i32),
             pltpu.VMEM((NCH, CHUNK), i32),
             pltpu.VMEM((nbuf, CHUNK, W), f32)]
            + [pltpu.SemaphoreType.DMA] * nbuf
            + [pltpu.VMEM_SHARED((SROWS, W), f32)]
        ),
    )


_seg = _make_seg(FT)

NCH1 = 79


@functools.partial(
    pl.kernel,
    out_type=jax.ShapeDtypeStruct((2 * SROWS,), f32),
    mesh=_mesh,
    scratch_types=[
        pltpu.VMEM((128,), f32),
        pltpu.VMEM((NCH1, CHUNK), i32),
        pltpu.VMEM((NCH, CHUNK), i32),
        pltpu.VMEM((NCH, CHUNK), i32),
        pltpu.VMEM((4, CHUNK), f32),
        pltpu.SemaphoreType.DMA,
        pltpu.SemaphoreType.DMA,
        pltpu.SemaphoreType.DMA,
        pltpu.SemaphoreType.DMA,
        pltpu.SemaphoreType.DMA,
        pltpu.SemaphoreType.DMA,
        pltpu.SemaphoreType.DMA,
        pltpu.SemaphoreType.DMA,
        pltpu.SemaphoreType.DMA,
        pltpu.VMEM_SHARED((SROWS,), f32),
        pltpu.VMEM_SHARED((SROWS,), f32),
    ],
)
def _deg(ones_in, s1e, gixe, sixn, z1, out, ov, s1v, gv, sv, gbuf,
         d0, g0, g1, g2, g3, s0, s1, s2, s3, aed, ad):
    cid = lax.axis_index("c")
    sid = lax.axis_index("s")
    wid = cid * NS + sid
    rpt = SROWS // NS
    pltpu.sync_copy(ones_in, ov)
    pltpu.sync_copy(s1e.at[sid], s1v)
    pltpu.sync_copy(gixe.at[wid], gv)
    pltpu.sync_copy(sixn.at[wid], sv)
    pltpu.sync_copy(z1, aed.at[pl.ds(sid * rpt, rpt)])
    pltpu.sync_copy(z1, ad.at[pl.ds(sid * rpt, rpt)])
    plsc.subcore_barrier()

    @pl.loop(0, NCH1)
    def _s1(ch):
        pltpu.async_copy(ov, aed.at[s1v.at[ch]], d0, add=True)

    @pl.loop(0, NCH1)
    def _s1d(ch):
        pltpu.make_async_copy(ov, aed.at[s1v.at[0]], d0).wait()

    plsc.subcore_barrier()
    gsem = [g0, g1, g2, g3]
    ssem = [s0, s1, s2, s3]
    pltpu.async_copy(aed.at[gv.at[0]], gbuf.at[0], gsem[0])
    pltpu.async_copy(aed.at[gv.at[1]], gbuf.at[1], gsem[1])

    @pl.loop(0, NCH, step=4)
    def _s2(c):
        for b in range(4):
            ch = c + b
            b2 = (b + 2) % 4
            pltpu.make_async_copy(aed.at[gv.at[ch]], gbuf.at[b],
                                  gsem[b]).wait()
            pltpu.async_copy(gbuf.at[b], ad.at[sv.at[ch]], ssem[b], add=True)

            @pl.when(ch + 2 < NCH)
            def _():
                @pl.when(ch >= 2)
                def _():
                    pltpu.make_async_copy(gbuf.at[b2], ad.at[sv.at[0]],
                                          ssem[b2]).wait()

                pltpu.async_copy(aed.at[gv.at[ch + 2]], gbuf.at[b2],
                                 gsem[b2])

    for b in range(4):
        pltpu.make_async_copy(gbuf.at[b], ad.at[sv.at[0]], ssem[b]).wait()
    plsc.subcore_barrier()
    pltpu.sync_copy(ad.at[pl.ds(sid * rpt, rpt)],
                    out.at[pl.ds(cid * SROWS + sid * rpt, rpt)])


BLK = 2000
GRID = N // BLK


def _row_spec(shape=(BLK, FT)):
    return pl.BlockSpec(shape, lambda i: (i, 0))


def _full_spec(shape):
    return pl.BlockSpec(shape, lambda i: (0, 0))


def _eadd(a, b):
    def body(ar, br, orf):
        orf[...] = ar[...] + br[...]

    spec = pl.BlockSpec((1024, FT), lambda i: (i, 0))
    return pl.pallas_call(
        body, grid=(SROWS // 1024,),
        in_specs=[spec, spec],
        out_specs=spec,
        out_shape=jax.ShapeDtypeStruct((SROWS, FT), f32),
    )(a, b)


def _bn_coeffs(str_, g, b):
    mu = str_[0:1, :] * (1.0 / N)
    var = str_[1:2, :] * (1.0 / N) - mu * mu
    scale = g * lax.rsqrt(var + 1e-5)
    shift = b - mu * scale
    return scale, shift


def _mlp(y0, y1, W1, b1, g1v, b1v, W2, b2, g2v, b2v):
    def body(y0r, y1r, w1r, b1r, g1r, bb1r, w2r, b2r, g2r, bb2r, outr,
             Ss, Ts, st1, st2):
        ph = pl.program_id(0)
        i = pl.program_id(1)
        rs = pl.ds(i * BLK, BLK)

        @pl.when(ph == 0)
        def _():
            @pl.when(i == 0)
            def _():
                st1[...] = jnp.zeros_like(st1)
                st2[...] = jnp.zeros_like(st2)

            x = y0r[...] + y1r[...]
            s = jnp.dot(x, w1r[...], preferred_element_type=f32) + b1r[...]
            Ss[rs, :] = s
            st1[0:1, :] += jnp.sum(s, axis=0, keepdims=True)
            st1[1:2, :] += jnp.sum(s * s, axis=0, keepdims=True)

        @pl.when(ph == 1)
        def _():
            sc, sh = _bn_coeffs(st1, g1r[...], bb1r[...])
            r = jnp.maximum(Ss[rs, :] * sc + sh, 0.0)
            t = jnp.dot(r, w2r[...], preferred_element_type=f32) + b2r[...]
            Ts[rs, :] = t
            st2[0:1, :] += jnp.sum(t, axis=0, keepdims=True)
            st2[1:2, :] += jnp.sum(t * t, axis=0, keepdims=True)

        @pl.when(ph == 2)
        def _():
            sc, sh = _bn_coeffs(st2, g2r[...], bb2r[...])
            outr[...] = jnp.maximum(Ts[rs, :] * sc + sh, 0.0)

    def first_phase(p, i):
        return (i * (p == 0), 0)

    return pl.pallas_call(
        body, grid=(3, GRID),
        in_specs=[pl.BlockSpec((BLK, FT), first_phase),
                  pl.BlockSpec((BLK, FT), first_phase),
                  pl.BlockSpec((FT, FT), lambda p, i: (0, 0)),
                  pl.BlockSpec((1, FT), lambda p, i: (0, 0)),
                  pl.BlockSpec((1, FT), lambda p, i: (0, 0)),
                  pl.BlockSpec((1, FT), lambda p, i: (0, 0)),
                  pl.BlockSpec((FT, FT), lambda p, i: (0, 0)),
                  pl.BlockSpec((1, FT), lambda p, i: (0, 0)),
                  pl.BlockSpec((1, FT), lambda p, i: (0, 0)),
                  pl.BlockSpec((1, FT), lambda p, i: (0, 0))],
        out_specs=pl.BlockSpec((BLK, FT), lambda p, i: (i, 0)),
        out_shape=jax.ShapeDtypeStruct((N, FT), f32),
        scratch_shapes=[pltpu.VMEM((N, FT), f32), pltpu.VMEM((N, FT), f32),
                        pltpu.VMEM((8, FT), f32), pltpu.VMEM((8, FT), f32)],
    )(y0, y1, W1, b1, g1v, b1v, W2, b2, g2v, b2v)


def _head(p00, p01, p10, p11, d0r, d1r, bidr, Wh0, Wh1, bh):
    def body(a0, a1, c0, c1, dr0, dr1, idsr, w0, w1, bhr, outr, accr, cntr):
        i = pl.program_id(0)
        dv = dr0[0, 0, :BLK] + dr1[0, 0, :BLK]
        dsafe = jnp.where(dv == 0.0, 1.0, dv)
        dneg = jnp.where(dv == 0.0, 1.0, 1.0 / dsafe)
        h0 = (a0[...] + a1[...]) * dneg[:, None]
        h1 = (c0[...] + c1[...]) * dneg[:, None]
        o = (jnp.dot(h0, w0[...], preferred_element_type=f32)
             + jnp.dot(h1, w1[...], preferred_element_type=f32) + bhr[...])
        ids = idsr[0, 0, :BLK]
        oh = (lax.broadcasted_iota(i32, (NG, BLK), 0) == ids[None, :]
              ).astype(f32)

        @pl.when(i == 0)
        def _():
            accr[...] = jnp.zeros_like(accr)
            cntr[...] = jnp.zeros_like(cntr)

        accr[...] += jnp.dot(oh, o, preferred_element_type=f32)
        cntr[:, 0:1] += jnp.sum(oh, axis=1, keepdims=True)
        outr[...] = accr[...] / jnp.maximum(cntr[:, 0:1], 1.0)

    return pl.pallas_call(
        body, grid=(GRID,),
        in_specs=[_row_spec(), _row_spec(), _row_spec(), _row_spec(),
                  pl.BlockSpec((1, 1, PADW), lambda i: (i, 0, 0)),
                  pl.BlockSpec((1, 1, PADW), lambda i: (i, 0, 0)),
                  pl.BlockSpec((1, 1, PADW), lambda i: (i, 0, 0)),
                  _full_spec((FT, NCLS)), _full_spec((FT, NCLS)),
                  _full_spec((1, NCLS))],
        out_specs=_full_spec((NG, NCLS)),
        out_shape=jax.ShapeDtypeStruct((NG, NCLS), f32),
        scratch_shapes=[pltpu.VMEM((NG, NCLS), f32),
                        pltpu.VMEM((NG, 8), f32)],
    )(p00, p01, p10, p11, d0r, d1r, bidr, Wh0, Wh1, bh)


def _pad_pairs(idx, pad):
    r = idx.reshape(NW, PPT)
    return jnp.concatenate([r, pad], axis=1).reshape(NW, NCH, CHUNK)


PADW = 2048


def _pad_rows(v):
    return jnp.pad(v.reshape(GRID, BLK),
                   ((0, 0), (0, PADW - BLK))).reshape(GRID, 1, PADW)


def kernel(X, params, H_node_idx, H_edge_idx, all_batch):
    p = params
    lane = jnp.arange(120, dtype=i32)[None, :]
    wrow = jnp.arange(NW, dtype=i32)[:, None]
    gpad = (lane * 97 + wrow * 53) % N
    spad = N + (lane * 7 + wrow * 11) % 240
    gp_n = _pad_pairs(H_node_idx, gpad)
    sp_n = _pad_pairs(H_node_idx, spad)
    gp_e = _pad_pairs(H_edge_idx, gpad)
    sp_e = _pad_pairs(H_edge_idx, spad)
    z2d = jnp.zeros((SROWS // NS, FT), f32)
    z1 = jnp.zeros((SROWS // NS,), f32)
    dpad = SROWS - 224 + (lane[:, :112] + wrow[:16]) % 64
    s1e = jnp.concatenate([H_edge_idx.reshape(NS, NNZ // NS), dpad],
                          axis=1).reshape(NS, NCH1, CHUNK)

    bidr = _pad_rows(all_batch)
    b1_0 = p['b1_0'].reshape(1, FT); b2_0 = p['b2_0'].reshape(1, FT)
    b1_1 = p['b1_1'].reshape(1, FT); b2_1 = p['b2_1'].reshape(1, FT)
    bng_0 = p['bng_0'].reshape(1, FT); bnb_0 = p['bnb_0'].reshape(1, FT)
    bng_1 = p['bng_1'].reshape(1, FT); bnb_1 = p['bnb_1'].reshape(1, FT)
    g_e0 = p['g_e0'].reshape(1, FT); b_e0 = p['b_e0'].reshape(1, FT)
    g_e1 = p['g_e1'].reshape(1, FT); b_e1 = p['b_e1'].reshape(1, FT)
    Wh0 = p['Wh'][:FT]; Wh1 = p['Wh'][FT:]
    bh = p['bh'].reshape(1, NCLS)

    def prop(src):
        ep = _seg(src, gp_n, sp_e, z2d)
        ef = _eadd(ep[0], ep[1])
        return _seg(ef, gp_e, sp_n, z2d)

    dflat = _deg(jnp.ones((CHUNK,), f32), s1e, gp_e, sp_n, z1)
    d0r = _pad_rows(dflat[:N])
    d1r = _pad_rows(dflat[SROWS:SROWS + N])

    y0 = prop(X)
    h0 = _mlp(y0[0], y0[1], p['W1_0'], b1_0, bng_0, bnb_0,
              p['W2_0'], b2_0, g_e0, b_e0)
    y1 = prop(h0)
    h1 = _mlp(y1[0], y1[1], p['W1_1'], b1_1, bng_1, bnb_1,
              p['W2_1'], b2_1, g_e1, b_e1)
    r1 = prop(h1)
    return _head(y1[0], y1[1], r1[0], r1[1], d0r, d1r, bidr, Wh0, Wh1, bh)
